# baseline (speedup 1.0000x reference)
import functools
import jax
import jax.numpy as jnp
from jax import lax
from jax.experimental import pallas as pl
from jax.experimental.pallas import tpu as pltpu

_EDGE = 64
_ANG = 64
_ATOM = 128
_RAD = 16
_NH = 2
_ISQ2 = 1.0 / 2.0 ** 0.5


def _bs(M, want):
    return want if M % want == 0 else M


def _sp(x):
    return jnp.maximum(x, 0.0) + jnp.log1p(jnp.exp(-jnp.abs(x)))


def _sig(x):
    return 1.0 / (1.0 + jnp.exp(-x))


def _residual(x, W1, b1, W2, b2):
    for h in range(_NH):
        x = x + jnp.dot(_sp(jnp.dot(x, W1[h], preferred_element_type=jnp.float32)
                            + b1[h]), W2[h],
                        preferred_element_type=jnp.float32) + b2[h]
    return x


def _full(shape):
    nd = len(shape)
    return pl.BlockSpec(shape, lambda i, _nd=nd: (0,) * _nd)


def _rows(bs, ncol):
    return pl.BlockSpec((bs, ncol), lambda i: (i, 0))


def _n2e_body(afs, afd, e, W, b, rW1, rb1, rW2, rb2, out):
    t = (jnp.dot(afs[...], W[:_ATOM], preferred_element_type=jnp.float32)
         + jnp.dot(afd[...], W[_ATOM:2 * _ATOM], preferred_element_type=jnp.float32)
         + jnp.dot(e[...], W[2 * _ATOM:], preferred_element_type=jnp.float32)
         + b[...])
    x = _sig(t[:, :_EDGE]) * _sp(t[:, _EDGE:])
    x = _residual(x, rW1[...], rb1[...], rW2[...], rb2[...])
    out[...] = _ISQ2 * (x + e[...])


def _n2e(afs, afd, edge, W, b, rW1, rb1, rW2, rb2):
    E = edge.shape[0]
    bs = _bs(E, 1000)
    return pl.pallas_call(
        _n2e_body,
        grid=(E // bs,),
        in_specs=[_rows(bs, _ATOM), _rows(bs, _ATOM), _rows(bs, _EDGE),
                  _full(W.shape), _full(b.shape), _full(rW1.shape),
                  _full(rb1.shape), _full(rW2.shape), _full(rb2.shape)],
        out_specs=_rows(bs, _EDGE),
        out_shape=jax.ShapeDtypeStruct((E, _EDGE), jnp.float32),
    )(afs, afd, edge, W, b, rW1, rb1, rW2, rb2)


def _e2a_body(ef0, ef1, ang, Wa, ba, rW1, rb1, rW2, rb2, Wm, bm,
              ang_out, msg_out):
    d00 = jnp.dot(ef0[...], Wa[:_EDGE], preferred_element_type=jnp.float32)
    d01 = jnp.dot(ef1[...], Wa[_EDGE:2 * _EDGE], preferred_element_type=jnp.float32)
    t = (d00 + d01
         + jnp.dot(ang[...], Wa[2 * _EDGE:], preferred_element_type=jnp.float32)
         + ba[...])
    x = _sig(t[:, :_ANG]) * _sp(t[:, _ANG:])
    x = _residual(x, rW1[...], rb1[...], rW2[...], rb2[...])
    af = _ISQ2 * (x + ang[...])
    ang_out[...] = af
    tm = (jnp.dot(ef0[...], Wm[:_EDGE], preferred_element_type=jnp.float32)
          + jnp.dot(ef1[...], Wm[_EDGE:2 * _EDGE], preferred_element_type=jnp.float32)
          + jnp.dot(af, Wm[2 * _EDGE:], preferred_element_type=jnp.float32)
          + bm[...])
    msg_out[...] = _sig(tm[:, :_EDGE]) * _sp(tm[:, _EDGE:])


def _e2a_a2e_msg(ef0, ef1, angle, Wa, ba, rW1, rb1, rW2, rb2, Wm, bm):
    A = angle.shape[0]
    bs = _bs(A, 1000)
    return pl.pallas_call(
        _e2a_body,
        grid=(A // bs,),
        in_specs=[_rows(bs, _EDGE), _rows(bs, _EDGE), _rows(bs, _ANG),
                  _full(Wa.shape), _full(ba.shape), _full(rW1.shape),
                  _full(rb1.shape), _full(rW2.shape), _full(rb2.shape),
                  _full(Wm.shape), _full(bm.shape)],
        out_specs=(_rows(bs, _ANG), _rows(bs, _EDGE)),
        out_shape=(jax.ShapeDtypeStruct((A, _ANG), jnp.float32),
                   jax.ShapeDtypeStruct((A, _EDGE), jnp.float32)),
    )(ef0, ef1, angle, Wa, ba, rW1, rb1, rW2, rb2, Wm, bm)


def _res_skip_body(agg, ori, rW1, rb1, rW2, rb2, out):
    x = _residual(agg[...], rW1[...], rb1[...], rW2[...], rb2[...])
    out[...] = _ISQ2 * (x + ori[...])


def _res_skip(agg, ori, rW1, rb1, rW2, rb2, want):
    M, D = agg.shape
    bs = _bs(M, want)
    return pl.pallas_call(
        _res_skip_body,
        grid=(M // bs,),
        in_specs=[_rows(bs, D), _rows(bs, D), _full(rW1.shape),
                  _full(rb1.shape), _full(rW2.shape), _full(rb2.shape)],
        out_specs=_rows(bs, D),
        out_shape=jax.ShapeDtypeStruct((M, D), jnp.float32),
    )(agg, ori, rW1, rb1, rW2, rb2)


def _res_skip2_body(agg0, agg1, ori, rW1, rb1, rW2, rb2, out):
    x = _residual(agg0[...] + agg1[...], rW1[...], rb1[...], rW2[...], rb2[...])
    out[...] = _ISQ2 * (x + ori[...])


def _res_skip2(agg0, agg1, ori, rW1, rb1, rW2, rb2, want):
    M, D = ori.shape
    bs = _bs(M, want)
    return pl.pallas_call(
        _res_skip2_body,
        grid=(M // bs,),
        in_specs=[_rows(bs, D), _rows(bs, D), _rows(bs, D), _full(rW1.shape),
                  _full(rb1.shape), _full(rW2.shape), _full(rb2.shape)],
        out_specs=_rows(bs, D),
        out_shape=jax.ShapeDtypeStruct((M, D), jnp.float32),
    )(agg0, agg1, ori, rW1, rb1, rW2, rb2)


def _e2n_body(afs, afd, ef, rbf, ats, atd, W, b, out):
    t = (jnp.dot(afs[...], W[:_ATOM], preferred_element_type=jnp.float32)
         + jnp.dot(afd[...], W[_ATOM:2 * _ATOM], preferred_element_type=jnp.float32)
         + jnp.dot(ef[...], W[2 * _ATOM:2 * _ATOM + _EDGE],
                   preferred_element_type=jnp.float32)
         + jnp.dot(rbf[...], W[2 * _ATOM + _EDGE:2 * _ATOM + _EDGE + _RAD],
                   preferred_element_type=jnp.float32)
         + jnp.dot(ats[:, :2], W[2 * _ATOM + _EDGE + _RAD:2 * _ATOM + _EDGE + _RAD + 2],
                   preferred_element_type=jnp.float32)
         + jnp.dot(atd[:, :2], W[2 * _ATOM + _EDGE + _RAD + 2:],
                   preferred_element_type=jnp.float32)
         + b[...])
    out[...] = _sig(t[:, :_ATOM]) * _sp(t[:, _ATOM:])


def _e2n_msg(afs, afd, ef, rbf, ats, atd, W, b):
    E = ef.shape[0]
    bs = _bs(E, 1000)
    return pl.pallas_call(
        _e2n_body,
        grid=(E // bs,),
        in_specs=[_rows(bs, _ATOM), _rows(bs, _ATOM), _rows(bs, _EDGE),
                  _rows(bs, _RAD), _rows(bs, 16), _rows(bs, 16),
                  _full(W.shape), _full(b.shape)],
        out_specs=_rows(bs, _ATOM),
        out_shape=jax.ShapeDtypeStruct((E, _ATOM), jnp.float32),
    )(afs, afd, ef, rbf, ats, atd, W, b)


from jax.experimental.pallas import tpu_sc as plsc

_NC = 2
_NS = 16
_NW = _NC * _NS


def _sc_gather(table, idx, chunk):
    B = idx.shape[0]
    D = table.shape[1]
    bpw = B // _NW
    nch = bpw // chunk
    mesh = plsc.VectorSubcoreMesh(core_axis_name="c", subcore_axis_name="s")

    @functools.partial(
        pl.kernel, mesh=mesh,
        compiler_params=pltpu.CompilerParams(use_tc_tiling_on_sc=False),
        out_type=jax.ShapeDtypeStruct((B, D), jnp.float32),
        scratch_types=[pltpu.VMEM((chunk,), jnp.int32),
                       pltpu.VMEM((chunk, D), jnp.float32),
                       pltpu.SemaphoreType.DMA],
    )
    def k(table_hbm, idx_hbm, out_hbm, idx_v, rows_v, sem):
        wid = lax.axis_index("s") * _NC + lax.axis_index("c")
        base = wid * bpw

        def body(i, carry):
            off = base + i * chunk
            pltpu.sync_copy(idx_hbm.at[pl.ds(off, chunk)], idx_v)
            pltpu.async_copy(table_hbm.at[idx_v], rows_v, sem).wait()
            pltpu.sync_copy(rows_v, out_hbm.at[pl.ds(off, chunk)])
            return carry

        lax.fori_loop(0, nch, body, 0)

    return k(table, idx)


def _sc_scatter_big(msg, idx, nrows, rng, chunk=800):
    A, D = msg.shape
    npass = nrows // (rng * _NC)
    apw = A // _NS
    nch = apw // chunk
    nvec = chunk // 16
    acc_rows = rng + 16
    rpt_z = acc_rows // _NS
    rpt_o = rng // _NS
    zeros = jnp.zeros((acc_rows, D), jnp.float32)
    mesh = plsc.VectorSubcoreMesh(core_axis_name="c", subcore_axis_name="s")

    @functools.partial(
        pl.kernel, mesh=mesh,
        compiler_params=pltpu.CompilerParams(use_tc_tiling_on_sc=False),
        out_type=jax.ShapeDtypeStruct((nrows, D), jnp.float32),
        scratch_types=[pltpu.VMEM((chunk,), jnp.int32),
                       pltpu.VMEM((chunk,), jnp.int32),
                       pltpu.VMEM((chunk, D), jnp.float32),
                       pltpu.VMEM_SHARED((acc_rows, D), jnp.float32)],
    )
    def k(msg_hbm, idx_hbm, zeros_hbm, out_hbm, idx_v, adj_v, msg_v, acc_sh):
        cid = lax.axis_index("c")
        sid = lax.axis_index("s")

        def dopass(p, carry0):
            rb = (p * _NC + cid) * rng
            pltpu.sync_copy(zeros_hbm.at[pl.ds(sid * rpt_z, rpt_z)],
                            acc_sh.at[pl.ds(sid * rpt_z, rpt_z)])
            plsc.subcore_barrier()

            def body(i, carry):
                off = sid * apw + i * chunk
                pltpu.sync_copy(idx_hbm.at[pl.ds(off, chunk)], idx_v)
                pltpu.sync_copy(msg_hbm.at[pl.ds(off, chunk)], msg_v)

                def adj(j, c2):
                    v = idx_v[pl.ds(j * 16, 16)]
                    inr = (v >= rb) & (v < rb + rng)
                    adj_v[pl.ds(j * 16, 16)] = jnp.where(inr, v - rb, rng)
                    return c2

                lax.fori_loop(0, nvec, adj, 0)
                pltpu.sync_copy(msg_v, acc_sh.at[adj_v], add=True)
                return carry

            lax.fori_loop(0, nch, body, 0)
            plsc.subcore_barrier()
            pltpu.sync_copy(acc_sh.at[pl.ds(sid * rpt_o, rpt_o)],
                            out_hbm.at[pl.ds(rb + sid * rpt_o, rpt_o)])
            plsc.subcore_barrier()
            return carry0

        lax.fori_loop(0, npass, dopass, 0)

    return k(msg, idx, zeros)


def _gather_rows(table, idx, chunk=None):
    B = idx.shape[0]
    D = table.shape[1]
    if chunk is None:
        chunk = 400 if D >= 128 else 1000
    if B % (_NW * chunk) == 0 and D % 16 == 0:
        return _sc_gather(table, jnp.asarray(idx, jnp.int32), chunk)
    return jnp.take(table, idx, axis=0)


def _scatter_add(msg, idx, nrows):
    B, D = msg.shape
    if B % (_NS * 800) == 0 and D % 16 == 0:
        idx = jnp.asarray(idx, jnp.int32)
        if D == 128:
            msg2 = msg.reshape(2 * B, 64)
            idx2 = jnp.stack([2 * idx, 2 * idx + 1], axis=1).reshape(-1)
            rng = 5120
            npad = -(-(2 * nrows) // (rng * _NC)) * (rng * _NC)
            out = _sc_scatter_big(msg2, idx2, npad, rng)
            return out[:2 * nrows].reshape(nrows, 128)
        rng = 16000
        npad = -(-nrows // (rng * _NC)) * (rng * _NC)
        out = _sc_scatter_big(msg, idx, npad, rng)
        return out[:nrows]
    return jnp.zeros((nrows, D), msg.dtype).at[idx].add(msg)


def kernel(atom_fea, edge, angle, rbf, atom_attr, W_n2e, b_n2e, W_e2a, b_e2a,
           W_a2e, b_a2e, W_e2n1, b_e2n1, W_e2n2, b_e2n2, er1_W1, er1_b1,
           er1_W2, er1_b2, ar2_W1, ar2_b1, ar2_W2, ar2_b2, er3_W1, er3_b1,
           er3_W2, er3_b2, nr1_W1, nr1_b1, nr1_W2, nr1_b2, nr2_W1, nr2_b1,
           nr2_W2, nr2_b2, nbr_fea_idx, nbr_swap_idx, angle_nbr_idx,
           crystal_edge_idx, crystal_angle_idx):
    N = atom_fea.shape[0]
    E = edge.shape[0]

    src = nbr_fea_idx[:, 0]
    dst = nbr_fea_idx[:, 1]
    a0 = angle_nbr_idx[:, 0]
    a1 = angle_nbr_idx[:, 1]
    b_n2e = b_n2e.reshape(1, -1)
    b_e2a = b_e2a.reshape(1, -1)
    b_a2e = b_a2e.reshape(1, -1)
    b_e2n1 = b_e2n1.reshape(1, -1)
    b_e2n2 = b_e2n2.reshape(1, -1)
    attr_pad = jnp.pad(atom_attr, ((0, 0), (0, 14)))

    afs = _gather_rows(atom_fea, src)
    afd = _gather_rows(atom_fea, dst)
    edge_fea = _n2e(afs, afd, edge, W_n2e, b_n2e, er1_W1, er1_b1, er1_W2,
                    er1_b2)

    ef0 = _gather_rows(edge_fea, a0)
    ef1 = _gather_rows(edge_fea, a1)
    angle_fea, msg = _e2a_a2e_msg(ef0, ef1, angle, W_e2a, b_e2a, ar2_W1,
                                  ar2_b1, ar2_W2, ar2_b2, W_a2e, b_a2e)
    agg = _scatter_add(msg, a0, E)
    edge_fea = _res_skip(agg, edge_fea, er3_W1, er3_b1, er3_W2, er3_b2, 1000)

    ats = _gather_rows(attr_pad, src)
    atd = _gather_rows(attr_pad, dst)
    msg = _e2n_msg(afs, afd, edge_fea, rbf, ats, atd, W_e2n1, b_e2n1)
    agg = _scatter_add(msg, dst, N)
    atom_fea1 = _res_skip(agg, atom_fea, nr1_W1, nr1_b1, nr1_W2, nr1_b2, 400)

    afs = _gather_rows(atom_fea1, src)
    afd = _gather_rows(atom_fea1, dst)
    msg = _e2n_msg(afs, afd, edge_fea, rbf, ats, atd, W_e2n2, b_e2n2)
    agg = _scatter_add(msg, dst, N)
    atom_fea2 = _res_skip(agg, atom_fea1, nr2_W1, nr2_b1, nr2_W2, nr2_b2, 400)

    return atom_fea2, edge_fea, angle_fea


# speedup vs baseline: 1.1114x; 1.1114x over previous
import functools
import jax
import jax.numpy as jnp
from jax import lax
from jax.experimental import pallas as pl
from jax.experimental.pallas import tpu as pltpu

_EDGE = 64
_ANG = 64
_ATOM = 128
_RAD = 16
_NH = 2
_ISQ2 = 1.0 / 2.0 ** 0.5


def _bs(M, want):
    return want if M % want == 0 else M


def _sp(x):
    return jnp.maximum(x, 0.0) + jnp.log1p(jnp.exp(-jnp.abs(x)))


def _sig(x):
    return 1.0 / (1.0 + jnp.exp(-x))


def _residual(x, W1, b1, W2, b2):
    for h in range(_NH):
        x = x + jnp.dot(_sp(jnp.dot(x, W1[h], preferred_element_type=jnp.float32)
                            + b1[h]), W2[h],
                        preferred_element_type=jnp.float32) + b2[h]
    return x


def _full(shape):
    nd = len(shape)
    return pl.BlockSpec(shape, lambda i, _nd=nd: (0,) * _nd)


def _rows(bs, ncol):
    return pl.BlockSpec((bs, ncol), lambda i: (i, 0))


def _n2e_body(afs, afd, e, W, b, rW1, rb1, rW2, rb2, out):
    t = (jnp.dot(afs[...], W[:_ATOM], preferred_element_type=jnp.float32)
         + jnp.dot(afd[...], W[_ATOM:2 * _ATOM], preferred_element_type=jnp.float32)
         + jnp.dot(e[...], W[2 * _ATOM:], preferred_element_type=jnp.float32)
         + b[...])
    x = _sig(t[:, :_EDGE]) * _sp(t[:, _EDGE:])
    x = _residual(x, rW1[...], rb1[...], rW2[...], rb2[...])
    out[...] = _ISQ2 * (x + e[...])


def _n2e(afs, afd, edge, W, b, rW1, rb1, rW2, rb2):
    E = edge.shape[0]
    bs = _bs(E, 1000)
    return pl.pallas_call(
        _n2e_body,
        grid=(E // bs,),
        in_specs=[_rows(bs, _ATOM), _rows(bs, _ATOM), _rows(bs, _EDGE),
                  _full(W.shape), _full(b.shape), _full(rW1.shape),
                  _full(rb1.shape), _full(rW2.shape), _full(rb2.shape)],
        out_specs=_rows(bs, _EDGE),
        out_shape=jax.ShapeDtypeStruct((E, _EDGE), jnp.float32),
    )(afs, afd, edge, W, b, rW1, rb1, rW2, rb2)


def _e2a_body(ef0, ef1, ang, Wa, ba, rW1, rb1, rW2, rb2, Wm, bm,
              ang_out, msg_out):
    d00 = jnp.dot(ef0[...], Wa[:_EDGE], preferred_element_type=jnp.float32)
    d01 = jnp.dot(ef1[...], Wa[_EDGE:2 * _EDGE], preferred_element_type=jnp.float32)
    t = (d00 + d01
         + jnp.dot(ang[...], Wa[2 * _EDGE:], preferred_element_type=jnp.float32)
         + ba[...])
    x = _sig(t[:, :_ANG]) * _sp(t[:, _ANG:])
    x = _residual(x, rW1[...], rb1[...], rW2[...], rb2[...])
    af = _ISQ2 * (x + ang[...])
    ang_out[...] = af
    tm = (jnp.dot(ef0[...], Wm[:_EDGE], preferred_element_type=jnp.float32)
          + jnp.dot(ef1[...], Wm[_EDGE:2 * _EDGE], preferred_element_type=jnp.float32)
          + jnp.dot(af, Wm[2 * _EDGE:], preferred_element_type=jnp.float32)
          + bm[...])
    msg_out[...] = _sig(tm[:, :_EDGE]) * _sp(tm[:, _EDGE:])


def _e2a_a2e_msg(ef0, ef1, angle, Wa, ba, rW1, rb1, rW2, rb2, Wm, bm):
    A = angle.shape[0]
    bs = _bs(A, 1000)
    return pl.pallas_call(
        _e2a_body,
        grid=(A // bs,),
        in_specs=[_rows(bs, _EDGE), _rows(bs, _EDGE), _rows(bs, _ANG),
                  _full(Wa.shape), _full(ba.shape), _full(rW1.shape),
                  _full(rb1.shape), _full(rW2.shape), _full(rb2.shape),
                  _full(Wm.shape), _full(bm.shape)],
        out_specs=(_rows(bs, _ANG), _rows(bs, _EDGE)),
        out_shape=(jax.ShapeDtypeStruct((A, _ANG), jnp.float32),
                   jax.ShapeDtypeStruct((A, _EDGE), jnp.float32)),
    )(ef0, ef1, angle, Wa, ba, rW1, rb1, rW2, rb2, Wm, bm)


def _res_skip_body(agg, ori, rW1, rb1, rW2, rb2, out):
    x = _residual(agg[...], rW1[...], rb1[...], rW2[...], rb2[...])
    out[...] = _ISQ2 * (x + ori[...])


def _res_skip(agg, ori, rW1, rb1, rW2, rb2, want):
    M, D = agg.shape
    bs = _bs(M, want)
    return pl.pallas_call(
        _res_skip_body,
        grid=(M // bs,),
        in_specs=[_rows(bs, D), _rows(bs, D), _full(rW1.shape),
                  _full(rb1.shape), _full(rW2.shape), _full(rb2.shape)],
        out_specs=_rows(bs, D),
        out_shape=jax.ShapeDtypeStruct((M, D), jnp.float32),
    )(agg, ori, rW1, rb1, rW2, rb2)


def _res_skip2_body(agg0, agg1, ori, rW1, rb1, rW2, rb2, out):
    x = _residual(agg0[...] + agg1[...], rW1[...], rb1[...], rW2[...], rb2[...])
    out[...] = _ISQ2 * (x + ori[...])


def _res_skip2(agg0, agg1, ori, rW1, rb1, rW2, rb2, want):
    M, D = ori.shape
    bs = _bs(M, want)
    return pl.pallas_call(
        _res_skip2_body,
        grid=(M // bs,),
        in_specs=[_rows(bs, D), _rows(bs, D), _rows(bs, D), _full(rW1.shape),
                  _full(rb1.shape), _full(rW2.shape), _full(rb2.shape)],
        out_specs=_rows(bs, D),
        out_shape=jax.ShapeDtypeStruct((M, D), jnp.float32),
    )(agg0, agg1, ori, rW1, rb1, rW2, rb2)


def _e2n_body(afs, afd, ef, rbf, ats, atd, W, b, out):
    t = (jnp.dot(afs[...], W[:_ATOM], preferred_element_type=jnp.float32)
         + jnp.dot(afd[...], W[_ATOM:2 * _ATOM], preferred_element_type=jnp.float32)
         + jnp.dot(ef[...], W[2 * _ATOM:2 * _ATOM + _EDGE],
                   preferred_element_type=jnp.float32)
         + jnp.dot(rbf[...], W[2 * _ATOM + _EDGE:2 * _ATOM + _EDGE + _RAD],
                   preferred_element_type=jnp.float32)
         + jnp.dot(ats[:, :2], W[2 * _ATOM + _EDGE + _RAD:2 * _ATOM + _EDGE + _RAD + 2],
                   preferred_element_type=jnp.float32)
         + jnp.dot(atd[:, :2], W[2 * _ATOM + _EDGE + _RAD + 2:],
                   preferred_element_type=jnp.float32)
         + b[...])
    out[...] = _sig(t[:, :_ATOM]) * _sp(t[:, _ATOM:])


def _e2n_msg(afs, afd, ef, rbf, ats, atd, W, b):
    E = ef.shape[0]
    bs = _bs(E, 1000)
    return pl.pallas_call(
        _e2n_body,
        grid=(E // bs,),
        in_specs=[_rows(bs, _ATOM), _rows(bs, _ATOM), _rows(bs, _EDGE),
                  _rows(bs, _RAD), _rows(bs, 16), _rows(bs, 16),
                  _full(W.shape), _full(b.shape)],
        out_specs=_rows(bs, _ATOM),
        out_shape=jax.ShapeDtypeStruct((E, _ATOM), jnp.float32),
    )(afs, afd, ef, rbf, ats, atd, W, b)


from jax.experimental.pallas import tpu_sc as plsc

_NC = 2
_NS = 16
_NW = _NC * _NS


def _sc_gather(table, idx, chunk):
    B = idx.shape[0]
    D = table.shape[1]
    bpw = B // _NW
    nch = bpw // chunk
    mesh = plsc.VectorSubcoreMesh(core_axis_name="c", subcore_axis_name="s")

    @functools.partial(
        pl.kernel, mesh=mesh,
        compiler_params=pltpu.CompilerParams(use_tc_tiling_on_sc=False),
        out_type=jax.ShapeDtypeStruct((B, D), jnp.float32),
        scratch_types=[pltpu.VMEM((chunk,), jnp.int32),
                       pltpu.VMEM((chunk,), jnp.int32),
                       pltpu.VMEM((chunk, D), jnp.float32),
                       pltpu.VMEM((chunk, D), jnp.float32),
                       pltpu.SemaphoreType.DMA,
                       pltpu.SemaphoreType.DMA,
                       pltpu.SemaphoreType.DMA,
                       pltpu.SemaphoreType.DMA],
    )
    def k(table_hbm, idx_hbm, out_hbm, idxA, idxB, rowsA, rowsB,
          gsA, gsB, wsA, wsB):
        wid = lax.axis_index("s") * _NC + lax.axis_index("c")
        base = wid * bpw

        def body(i, carry):
            offA = base + 2 * i * chunk
            offB = offA + chunk
            pltpu.sync_copy(idx_hbm.at[pl.ds(offA, chunk)], idxA)
            gA = pltpu.async_copy(table_hbm.at[idxA], rowsA, gsA)
            pltpu.sync_copy(idx_hbm.at[pl.ds(offB, chunk)], idxB)
            gB = pltpu.async_copy(table_hbm.at[idxB], rowsB, gsB)
            gA.wait()
            wA = pltpu.async_copy(rowsA, out_hbm.at[pl.ds(offA, chunk)], wsA)
            gB.wait()
            wB = pltpu.async_copy(rowsB, out_hbm.at[pl.ds(offB, chunk)], wsB)
            wA.wait()
            wB.wait()
            return carry

        lax.fori_loop(0, nch // 2, body, 0)

    return k(table, idx)


def _sc_scatter_big(msg, idx, nrows, rng, chunk=800):
    A, D = msg.shape
    npass = nrows // (rng * _NC)
    apw = A // _NS
    nch = apw // chunk
    nvec = chunk // 16
    acc_rows = rng + 16
    rpt_z = acc_rows // _NS
    rpt_o = rng // _NS
    zeros = jnp.zeros((acc_rows, D), jnp.float32)
    mesh = plsc.VectorSubcoreMesh(core_axis_name="c", subcore_axis_name="s")

    @functools.partial(
        pl.kernel, mesh=mesh,
        compiler_params=pltpu.CompilerParams(use_tc_tiling_on_sc=False),
        out_type=jax.ShapeDtypeStruct((nrows, D), jnp.float32),
        scratch_types=[pltpu.VMEM((chunk,), jnp.int32),
                       pltpu.VMEM((chunk,), jnp.int32),
                       pltpu.VMEM((chunk, D), jnp.float32),
                       pltpu.VMEM_SHARED((acc_rows, D), jnp.float32)],
    )
    def k(msg_hbm, idx_hbm, zeros_hbm, out_hbm, idx_v, adj_v, msg_v, acc_sh):
        cid = lax.axis_index("c")
        sid = lax.axis_index("s")

        def dopass(p, carry0):
            rb = (p * _NC + cid) * rng
            pltpu.sync_copy(zeros_hbm.at[pl.ds(sid * rpt_z, rpt_z)],
                            acc_sh.at[pl.ds(sid * rpt_z, rpt_z)])
            plsc.subcore_barrier()

            def body(i, carry):
                off = sid * apw + i * chunk
                pltpu.sync_copy(idx_hbm.at[pl.ds(off, chunk)], idx_v)
                pltpu.sync_copy(msg_hbm.at[pl.ds(off, chunk)], msg_v)

                def adj(j, c2):
                    v = idx_v[pl.ds(j * 16, 16)]
                    inr = (v >= rb) & (v < rb + rng)
                    adj_v[pl.ds(j * 16, 16)] = jnp.where(inr, v - rb, rng)
                    return c2

                lax.fori_loop(0, nvec, adj, 0)
                pltpu.sync_copy(msg_v, acc_sh.at[adj_v], add=True)
                return carry

            lax.fori_loop(0, nch, body, 0)
            plsc.subcore_barrier()
            pltpu.sync_copy(acc_sh.at[pl.ds(sid * rpt_o, rpt_o)],
                            out_hbm.at[pl.ds(rb + sid * rpt_o, rpt_o)])
            plsc.subcore_barrier()
            return carry0

        lax.fori_loop(0, npass, dopass, 0)

    return k(msg, idx, zeros)


def _gather_rows(table, idx, chunk=None):
    B = idx.shape[0]
    D = table.shape[1]
    if chunk is None:
        chunk = 1000 if D <= 16 else 200
    if B % (_NW * chunk * 2) == 0 and D % 16 == 0:
        return _sc_gather(table, jnp.asarray(idx, jnp.int32), chunk)
    return jnp.take(table, idx, axis=0)


def _scatter_add(msg, idx, nrows):
    B, D = msg.shape
    if B % (_NS * 800) == 0 and D % 16 == 0:
        rng = 2560 if D == 128 else 16000
        npad = -(-nrows // (rng * _NC)) * (rng * _NC)
        out = _sc_scatter_big(msg, jnp.asarray(idx, jnp.int32), npad, rng)
        return out[:nrows]
    return jnp.zeros((nrows, D), msg.dtype).at[idx].add(msg)


def kernel(atom_fea, edge, angle, rbf, atom_attr, W_n2e, b_n2e, W_e2a, b_e2a,
           W_a2e, b_a2e, W_e2n1, b_e2n1, W_e2n2, b_e2n2, er1_W1, er1_b1,
           er1_W2, er1_b2, ar2_W1, ar2_b1, ar2_W2, ar2_b2, er3_W1, er3_b1,
           er3_W2, er3_b2, nr1_W1, nr1_b1, nr1_W2, nr1_b2, nr2_W1, nr2_b1,
           nr2_W2, nr2_b2, nbr_fea_idx, nbr_swap_idx, angle_nbr_idx,
           crystal_edge_idx, crystal_angle_idx):
    N = atom_fea.shape[0]
    E = edge.shape[0]

    src = nbr_fea_idx[:, 0]
    dst = nbr_fea_idx[:, 1]
    a0 = angle_nbr_idx[:, 0]
    a1 = angle_nbr_idx[:, 1]
    b_n2e = b_n2e.reshape(1, -1)
    b_e2a = b_e2a.reshape(1, -1)
    b_a2e = b_a2e.reshape(1, -1)
    b_e2n1 = b_e2n1.reshape(1, -1)
    b_e2n2 = b_e2n2.reshape(1, -1)
    attr_pad = jnp.pad(atom_attr, ((0, 0), (0, 14)))

    afs = _gather_rows(atom_fea, src)
    afd = _gather_rows(atom_fea, dst)
    edge_fea = _n2e(afs, afd, edge, W_n2e, b_n2e, er1_W1, er1_b1, er1_W2,
                    er1_b2)

    ef0 = _gather_rows(edge_fea, a0)
    ef1 = _gather_rows(edge_fea, a1)
    angle_fea, msg = _e2a_a2e_msg(ef0, ef1, angle, W_e2a, b_e2a, ar2_W1,
                                  ar2_b1, ar2_W2, ar2_b2, W_a2e, b_a2e)
    agg = _scatter_add(msg, a0, E)
    edge_fea = _res_skip(agg, edge_fea, er3_W1, er3_b1, er3_W2, er3_b2, 1000)

    ats = _gather_rows(attr_pad, src)
    atd = _gather_rows(attr_pad, dst)
    msg = _e2n_msg(afs, afd, edge_fea, rbf, ats, atd, W_e2n1, b_e2n1)
    agg = _scatter_add(msg, dst, N)
    atom_fea1 = _res_skip(agg, atom_fea, nr1_W1, nr1_b1, nr1_W2, nr1_b2, 400)

    afs = _gather_rows(atom_fea1, src)
    afd = _gather_rows(atom_fea1, dst)
    msg = _e2n_msg(afs, afd, edge_fea, rbf, ats, atd, W_e2n2, b_e2n2)
    agg = _scatter_add(msg, dst, N)
    atom_fea2 = _res_skip(agg, atom_fea1, nr2_W1, nr2_b1, nr2_W2, nr2_b2, 400)

    return atom_fea2, edge_fea, angle_fea
